# baseline (speedup 1.0000x reference)
import functools
import math

import jax
import jax.numpy as jnp
from jax import lax
from jax.experimental import pallas as pl
from jax.experimental.pallas import tpu as pltpu
from jax.experimental.pallas import tpu_sc as plsc

VOCAB = 1000000
EMBED = 64
BATCH = 4096
SEQ = 200
B = BATCH * SEQ

L = 16
NC, NS = 2, 16
NW = NC * NS
B_PER_W = B // NW
SUB = 128
CHUNK = 512
NSUB = CHUNK // SUB
NCHUNK = B_PER_W // CHUNK
IDXROWS_PER_W = B_PER_W // SUB
SCALE = math.sqrt(EMBED)


def _emb_kernel(idx_hbm, tab_hbm, out_hbm, idx_v, rows_v, sem):
    wid = lax.axis_index("s") * NC + lax.axis_index("c")
    out_base = wid * B_PER_W
    idx_base = wid * IDXROWS_PER_W

    def chunk_body(c, carry):
        pltpu.sync_copy(idx_hbm.at[pl.ds(idx_base + c * NSUB, NSUB)], idx_v)
        copies = []
        for j in range(NSUB):
            copies.append(pltpu.async_copy(
                tab_hbm.at[idx_v.at[j]],
                rows_v.at[pl.ds(j * SUB, SUB)],
                sem,
            ))
        for cp in copies:
            cp.wait()

        def row_body(i, carry2):
            for j in range(EMBED // L):
                rows_v[i, pl.ds(j * L, L)] = rows_v[i, pl.ds(j * L, L)] * SCALE
            return carry2
        lax.fori_loop(0, CHUNK, row_body, 0, unroll=4)

        pltpu.sync_copy(rows_v, out_hbm.at[pl.ds(out_base + c * CHUNK, CHUNK)])
        return carry

    lax.fori_loop(0, NCHUNK, chunk_body, 0)


@jax.jit
def _emb(idx2d, table):
    mesh = plsc.VectorSubcoreMesh(core_axis_name="c", subcore_axis_name="s")
    return pl.kernel(
        _emb_kernel,
        mesh=mesh,
        out_type=jax.ShapeDtypeStruct((B, EMBED), jnp.float32),
        scratch_types=[
            pltpu.VMEM((NSUB, SUB), jnp.int32),
            pltpu.VMEM((CHUNK, EMBED), jnp.float32),
            pltpu.SemaphoreType.DMA,
        ],
        compiler_params=pltpu.CompilerParams(use_tc_tiling_on_sc=False),
    )(idx2d, table)


def kernel(inputs, table):
    idx2d = inputs.reshape(B // SUB, SUB)
    out = _emb(idx2d, table)
    return out.reshape(BATCH, SEQ, EMBED)


# speedup vs baseline: 1.0766x; 1.0766x over previous
import functools
import math

import jax
import jax.numpy as jnp
from jax import lax
from jax.experimental import pallas as pl
from jax.experimental.pallas import tpu as pltpu
from jax.experimental.pallas import tpu_sc as plsc

VOCAB = 1000000
EMBED = 64
BATCH = 4096
SEQ = 200
B = BATCH * SEQ

L = 16
NC, NS = 2, 16
NW = NC * NS
B_PER_W = B // NW
SUB = 128
CHUNK = 512
NSUB = CHUNK // SUB
NCHUNK = B_PER_W // CHUNK
IDXROWS_PER_W = B_PER_W // SUB
SCALE = math.sqrt(EMBED)


def _emb_kernel(idx_hbm, tab_hbm, out_hbm,
                idx0, idx1, rows0, rows1, sem0, sem1):
    wid = lax.axis_index("s") * NC + lax.axis_index("c")
    out_base = wid * B_PER_W
    idx_base = wid * IDXROWS_PER_W
    idx_v = (idx0, idx1)
    rows_v = (rows0, rows1)
    sems = (sem0, sem1)

    def fire(b, c):
        pltpu.sync_copy(idx_hbm.at[pl.ds(idx_base + c * NSUB, NSUB)], idx_v[b])
        for j in range(NSUB):
            pltpu.async_copy(
                tab_hbm.at[idx_v[b].at[j]],
                rows_v[b].at[pl.ds(j * SUB, SUB)],
                sems[b],
            )

    def drain(b):
        for j in range(NSUB):
            pltpu.make_async_copy(
                tab_hbm.at[idx_v[b].at[j]],
                rows_v[b].at[pl.ds(j * SUB, SUB)],
                sems[b],
            ).wait()

    def scale(b):
        rows = rows_v[b]

        @plsc.parallel_loop(0, CHUNK, step=1, unroll=8)
        def _(i):
            for j in range(EMBED // L):
                rows[i, pl.ds(j * L, L)] = rows[i, pl.ds(j * L, L)] * SCALE

    for b in range(2):
        fire(b, b)

    def group_body(g, carry):
        for b in range(2):
            c = g * 2 + b
            drain(b)
            scale(b)
            pltpu.sync_copy(rows_v[b], out_hbm.at[pl.ds(out_base + c * CHUNK, CHUNK)])

            @pl.when(c + 2 < NCHUNK)
            def _():
                fire(b, c + 2)
        return carry

    lax.fori_loop(0, NCHUNK // 2, group_body, 0)


@jax.jit
def _emb(idx2d, table):
    mesh = plsc.VectorSubcoreMesh(core_axis_name="c", subcore_axis_name="s")
    return pl.kernel(
        _emb_kernel,
        mesh=mesh,
        out_type=jax.ShapeDtypeStruct((B, EMBED), jnp.float32),
        scratch_types=[
            pltpu.VMEM((NSUB, SUB), jnp.int32),
            pltpu.VMEM((NSUB, SUB), jnp.int32),
            pltpu.VMEM((CHUNK, EMBED), jnp.float32),
            pltpu.VMEM((CHUNK, EMBED), jnp.float32),
            pltpu.SemaphoreType.DMA,
            pltpu.SemaphoreType.DMA,
        ],
        compiler_params=pltpu.CompilerParams(use_tc_tiling_on_sc=False),
    )(idx2d, table)


def kernel(inputs, table):
    idx2d = inputs.reshape(B // SUB, SUB)
    out = _emb(idx2d, table)
    return out.reshape(BATCH, SEQ, EMBED)


# speedup vs baseline: 1.3093x; 1.2161x over previous
import functools
import math

import jax
import jax.numpy as jnp
from jax import lax
from jax.experimental import pallas as pl
from jax.experimental.pallas import tpu as pltpu
from jax.experimental.pallas import tpu_sc as plsc

VOCAB = 1000000
EMBED = 64
EPAD = 128
BATCH = 4096
SEQ = 200
B = BATCH * SEQ

L = 16
NC, NS = 2, 16
NW = NC * NS
B_PER_W = B // NW
SUB = 128
CHUNK = 256
NSUB = CHUNK // SUB
NCHUNK = B_PER_W // CHUNK
IDXROWS_PER_W = B_PER_W // SUB
SCALE = math.sqrt(EMBED)


def _emb_kernel(idx_hbm, tab_hbm, out_hbm,
                idx0, idx1, rows0, rows1, cmp_v, sem0, sem1):
    wid = lax.axis_index("s") * NC + lax.axis_index("c")
    out_base = wid * B_PER_W
    idx_base = wid * IDXROWS_PER_W
    idx_v = (idx0, idx1)
    rows_v = (rows0, rows1)
    sems = (sem0, sem1)

    def fire(b, c):
        pltpu.sync_copy(idx_hbm.at[pl.ds(idx_base + c * NSUB, NSUB)], idx_v[b])
        for j in range(NSUB):
            pltpu.async_copy(
                tab_hbm.at[idx_v[b].at[j]],
                rows_v[b].at[pl.ds(j * SUB, SUB)],
                sems[b],
            )

    def drain(b):
        for j in range(NSUB):
            pltpu.make_async_copy(
                tab_hbm.at[idx_v[b].at[j]],
                rows_v[b].at[pl.ds(j * SUB, SUB)],
                sems[b],
            ).wait()

    def scale(b):
        rows = rows_v[b]

        @plsc.parallel_loop(0, CHUNK, step=1, unroll=8)
        def _(i):
            for j in range(EMBED // L):
                cmp_v[i, pl.ds(j * L, L)] = rows[i, pl.ds(j * L, L)] * SCALE

    for b in range(2):
        fire(b, b)

    def group_body(g, carry):
        for b in range(2):
            c = g * 2 + b
            drain(b)
            scale(b)

            @pl.when(c + 2 < NCHUNK)
            def _():
                fire(b, c + 2)

            pltpu.sync_copy(cmp_v, out_hbm.at[pl.ds(out_base + c * CHUNK, CHUNK)])
        return carry

    lax.fori_loop(0, NCHUNK // 2, group_body, 0)


@jax.jit
def _emb(idx2d, tpad):
    mesh = plsc.VectorSubcoreMesh(core_axis_name="c", subcore_axis_name="s")
    return pl.kernel(
        _emb_kernel,
        mesh=mesh,
        out_type=jax.ShapeDtypeStruct((B, EMBED), jnp.float32),
        scratch_types=[
            pltpu.VMEM((NSUB, SUB), jnp.int32),
            pltpu.VMEM((NSUB, SUB), jnp.int32),
            pltpu.VMEM((CHUNK, EPAD), jnp.float32),
            pltpu.VMEM((CHUNK, EPAD), jnp.float32),
            pltpu.VMEM((CHUNK, EMBED), jnp.float32),
            pltpu.SemaphoreType.DMA,
            pltpu.SemaphoreType.DMA,
        ],
        compiler_params=pltpu.CompilerParams(use_tc_tiling_on_sc=True),
    )(idx2d, tpad)


def kernel(inputs, table):
    idx2d = inputs.reshape(B // SUB, SUB)
    tpad = jnp.pad(table, ((0, 0), (0, EPAD - EMBED)))
    out = _emb(idx2d, tpad)
    return out.reshape(BATCH, SEQ, EMBED)
